# baseline (speedup 1.0000x reference)
import functools

import jax
import jax.numpy as jnp
from jax import lax
from jax.experimental import pallas as pl
from jax.experimental.pallas import tpu as pltpu
from jax.experimental.pallas import tpu_sc as plsc

_B = 1024
_LCTX = 20
_E = 300
_EP = 384
_V = 100000

_NC = 2
_NS = 16
_NW = _NC * _NS
_ROWS = _B * _LCTX
_CHUNK = 128
_CHUNKS_PER_W = _ROWS // (_NW * _CHUNK)


def _sc_gather_body(idx_hbm, table_hbm, tail_hbm, out_hbm,
                    idx_v, p0_v, p1_v, p2_v, sem):
    wid = lax.axis_index("s") * _NC + lax.axis_index("c")
    crow = wid * _CHUNKS_PER_W
    pltpu.sync_copy(idx_hbm.at[wid], idx_v)
    for j in range(_CHUNKS_PER_W):
        row = idx_v.at[j]
        c0 = pltpu.async_copy(table_hbm.at[row, pl.ds(0, 128)], p0_v, sem)
        c1 = pltpu.async_copy(table_hbm.at[row, pl.ds(128, 128)], p1_v, sem)
        c2 = pltpu.async_copy(tail_hbm.at[row], p2_v, sem)
        c0.wait()
        c1.wait()
        c2.wait()
        dst = out_hbm.at[pl.ds((crow + j) * _CHUNK, _CHUNK)]
        pltpu.sync_copy(p0_v, dst.at[:, pl.ds(0, 128)])
        pltpu.sync_copy(p1_v, dst.at[:, pl.ds(128, 128)])
        pltpu.sync_copy(p2_v, dst.at[:, pl.ds(256, 128)])


@functools.cache
def _sc_gather():
    return pl.kernel(
        _sc_gather_body,
        out_type=jax.ShapeDtypeStruct((_ROWS, _EP), jnp.float32),
        mesh=plsc.VectorSubcoreMesh(core_axis_name="c", subcore_axis_name="s"),
        scratch_types=[
            pltpu.VMEM((_CHUNKS_PER_W, _CHUNK), jnp.int32),
            pltpu.VMEM((_CHUNK, 128), jnp.float32),
            pltpu.VMEM((_CHUNK, 128), jnp.float32),
            pltpu.VMEM((_CHUNK, 128), jnp.float32),
            pltpu.SemaphoreType.DMA,
        ],
    )


_RB = 4000


def _tail_body(t_ref, o_ref):
    o_ref[...] = t_ref[...]


_tail_pad = pl.pallas_call(
    _tail_body,
    grid=(_V // _RB,),
    in_specs=[pl.BlockSpec((_RB, 128), lambda i: (i, 2))],
    out_specs=pl.BlockSpec((_RB, 128), lambda i: (i, 0)),
    out_shape=jax.ShapeDtypeStruct((_V, 128), jnp.float32),
    compiler_params=pltpu.CompilerParams(
        dimension_semantics=("arbitrary",),
    ),
)


_BB = 128


def _pool_body(emb_ref, x_ref):
    emb = emb_ref[...]
    e = emb[:, :, :_E]
    n2 = jnp.sum(e * e, axis=-1, keepdims=True)
    scale = jnp.where(n2 > 1.0, lax.rsqrt(n2), 1.0)
    x_ref[...] = jnp.mean(e * scale, axis=1)


_pool = pl.pallas_call(
    _pool_body,
    grid=(_B // _BB,),
    in_specs=[pl.BlockSpec((_BB, _LCTX, _EP), lambda i: (i, 0, 0))],
    out_specs=pl.BlockSpec((_BB, _E), lambda i: (i, 0)),
    out_shape=jax.ShapeDtypeStruct((_B, _E), jnp.float32),
)


_BN = 4096
_NBLK = pl.cdiv(_V, _BN)
_LAST = _V - (_NBLK - 1) * _BN
_NSLOT = 2


def _proj_body(x_ref, w_ref, b_ref, o_hbm, buf, buf_last, sems):
    i = pl.program_id(0)
    slot = lax.rem(i, _NSLOT)

    def fullcopy(s, blkidx):
        return pltpu.make_async_copy(
            buf.at[s], o_hbm.at[:, pl.ds(blkidx * _BN, _BN)], sems.at[s])

    def lastcopy(s):
        return pltpu.make_async_copy(
            buf_last, o_hbm.at[:, pl.ds((_NBLK - 1) * _BN, _LAST)], sems.at[s])

    @pl.when(i >= _NSLOT)
    def _wait():
        fullcopy(slot, i - _NSLOT).wait()

    acc = lax.dot_general(x_ref[...], w_ref[...], (((1,), (1,)), ((), ())),
                          preferred_element_type=jnp.float32)
    buf[slot] = acc + b_ref[...]

    @pl.when(i < _NBLK - 1)
    def _issue():
        fullcopy(slot, i).start()

    @pl.when(i == _NBLK - 1)
    def _finish():
        buf_last[...] = buf[slot, :, :_LAST]
        lastcopy(slot).start()
        for k in range(_NSLOT - 1):
            blkidx = _NBLK - _NSLOT + k
            fullcopy(blkidx % _NSLOT, blkidx).wait()
        lastcopy((_NBLK - 1) % _NSLOT).wait()


_proj = pl.pallas_call(
    _proj_body,
    grid=(_NBLK,),
    in_specs=[
        pl.BlockSpec((_B, _E), lambda i: (0, 0)),
        pl.BlockSpec((_BN, _E), lambda i: (i, 0)),
        pl.BlockSpec((1, _BN), lambda i: (0, i)),
    ],
    out_specs=pl.BlockSpec(memory_space=pl.ANY),
    out_shape=jax.ShapeDtypeStruct((_B, _V), jnp.float32),
    scratch_shapes=[pltpu.VMEM((_NSLOT, _B, _BN), jnp.float32),
                    pltpu.VMEM((_B, _LAST), jnp.float32),
                    pltpu.SemaphoreType.DMA((_NSLOT,))],
    compiler_params=pltpu.CompilerParams(
        dimension_semantics=("arbitrary",),
    ),
)


def kernel(inputs_, table, W, b):
    idx = inputs_.reshape(_NW, _CHUNKS_PER_W, _CHUNK).astype(jnp.int32)
    tail = _tail_pad(table)
    emb = _sc_gather()(idx, table, tail)
    x = _pool(emb.reshape(_B, _LCTX, _EP))
    return _proj(x, W, b.reshape(1, _V))


# speedup vs baseline: 1.0038x; 1.0038x over previous
import functools

import jax
import jax.numpy as jnp
from jax import lax
from jax.experimental import pallas as pl
from jax.experimental.pallas import tpu as pltpu
from jax.experimental.pallas import tpu_sc as plsc

_B = 1024
_LCTX = 20
_E = 300
_EP = 384
_V = 100000

_NC = 2
_NS = 16
_NW = _NC * _NS
_ROWS = _B * _LCTX
_CHUNK = 128
_CHUNKS_PER_W = _ROWS // (_NW * _CHUNK)


def _sc_gather_body(idx_hbm, table_hbm, tail_hbm, out_hbm,
                    idx_v, p0_v, p1_v, p2_v, q0_v, q1_v, q2_v, sem):
    wid = lax.axis_index("s") * _NC + lax.axis_index("c")
    crow = wid * _CHUNKS_PER_W
    pltpu.sync_copy(idx_hbm.at[wid], idx_v)
    bufs = [(p0_v, p1_v, p2_v), (q0_v, q1_v, q2_v)]

    def start(j, bs):
        row = idx_v.at[j]
        return (
            pltpu.async_copy(table_hbm.at[row, pl.ds(0, 128)], bs[0], sem),
            pltpu.async_copy(table_hbm.at[row, pl.ds(128, 128)], bs[1], sem),
            pltpu.async_copy(tail_hbm.at[row], bs[2], sem),
        )

    cs = start(0, bufs[0])
    for j in range(_CHUNKS_PER_W):
        nxt = start(j + 1, bufs[(j + 1) % 2]) if j + 1 < _CHUNKS_PER_W else None
        for c in cs:
            c.wait()
        bs = bufs[j % 2]
        dst = out_hbm.at[pl.ds((crow + j) * _CHUNK, _CHUNK)]
        pltpu.sync_copy(bs[0], dst.at[:, pl.ds(0, 128)])
        pltpu.sync_copy(bs[1], dst.at[:, pl.ds(128, 128)])
        pltpu.sync_copy(bs[2], dst.at[:, pl.ds(256, 128)])
        cs = nxt


@functools.cache
def _sc_gather():
    return pl.kernel(
        _sc_gather_body,
        out_type=jax.ShapeDtypeStruct((_ROWS, _EP), jnp.float32),
        mesh=plsc.VectorSubcoreMesh(core_axis_name="c", subcore_axis_name="s"),
        scratch_types=[
            pltpu.VMEM((_CHUNKS_PER_W, _CHUNK), jnp.int32),
            pltpu.VMEM((_CHUNK, 128), jnp.float32),
            pltpu.VMEM((_CHUNK, 128), jnp.float32),
            pltpu.VMEM((_CHUNK, 128), jnp.float32),
            pltpu.VMEM((_CHUNK, 128), jnp.float32),
            pltpu.VMEM((_CHUNK, 128), jnp.float32),
            pltpu.VMEM((_CHUNK, 128), jnp.float32),
            pltpu.SemaphoreType.DMA,
        ],
    )


_RB = 10000


def _tail_body(t_ref, o_ref):
    o_ref[...] = t_ref[...]


_tail_pad = pl.pallas_call(
    _tail_body,
    grid=(_V // _RB,),
    in_specs=[pl.BlockSpec((_RB, 128), lambda i: (i, 2))],
    out_specs=pl.BlockSpec((_RB, 128), lambda i: (i, 0)),
    out_shape=jax.ShapeDtypeStruct((_V, 128), jnp.float32),
    compiler_params=pltpu.CompilerParams(
        dimension_semantics=("arbitrary",),
    ),
)


_BB = 256


def _pool_body(emb_ref, x_ref):
    emb = emb_ref[...]
    e = emb[:, :, :_E]
    n2 = jnp.sum(e * e, axis=-1, keepdims=True)
    scale = jnp.where(n2 > 1.0, lax.rsqrt(n2), 1.0)
    x_ref[...] = jnp.mean(e * scale, axis=1)


_pool = pl.pallas_call(
    _pool_body,
    grid=(_B // _BB,),
    in_specs=[pl.BlockSpec((_BB, _LCTX, _EP), lambda i: (i, 0, 0))],
    out_specs=pl.BlockSpec((_BB, _E), lambda i: (i, 0)),
    out_shape=jax.ShapeDtypeStruct((_B, _E), jnp.float32),
)


_BN = 4096
_NBLK = pl.cdiv(_V, _BN)
_LAST = _V - (_NBLK - 1) * _BN
_NSLOT = 2


def _proj_body(x_ref, w_ref, b_ref, o_hbm, buf, buf_last, sems):
    i = pl.program_id(0)
    slot = lax.rem(i, _NSLOT)

    def fullcopy(s, blkidx):
        return pltpu.make_async_copy(
            buf.at[s], o_hbm.at[:, pl.ds(blkidx * _BN, _BN)], sems.at[s])

    def lastcopy(s):
        return pltpu.make_async_copy(
            buf_last, o_hbm.at[:, pl.ds((_NBLK - 1) * _BN, _LAST)], sems.at[s])

    @pl.when(i >= _NSLOT)
    def _wait():
        fullcopy(slot, i - _NSLOT).wait()

    acc = lax.dot_general(x_ref[...], w_ref[...], (((1,), (1,)), ((), ())),
                          preferred_element_type=jnp.float32)
    buf[slot] = acc + b_ref[...]

    @pl.when(i < _NBLK - 1)
    def _issue():
        fullcopy(slot, i).start()

    @pl.when(i == _NBLK - 1)
    def _finish():
        buf_last[...] = buf[slot, :, :_LAST]
        lastcopy(slot).start()
        for k in range(_NSLOT - 1):
            blkidx = _NBLK - _NSLOT + k
            fullcopy(blkidx % _NSLOT, blkidx).wait()
        lastcopy((_NBLK - 1) % _NSLOT).wait()


_proj = pl.pallas_call(
    _proj_body,
    grid=(_NBLK,),
    in_specs=[
        pl.BlockSpec((_B, _E), lambda i: (0, 0)),
        pl.BlockSpec((_BN, _E), lambda i: (i, 0)),
        pl.BlockSpec((1, _BN), lambda i: (0, i)),
    ],
    out_specs=pl.BlockSpec(memory_space=pl.ANY),
    out_shape=jax.ShapeDtypeStruct((_B, _V), jnp.float32),
    scratch_shapes=[pltpu.VMEM((_NSLOT, _B, _BN), jnp.float32),
                    pltpu.VMEM((_B, _LAST), jnp.float32),
                    pltpu.SemaphoreType.DMA((_NSLOT,))],
    compiler_params=pltpu.CompilerParams(
        dimension_semantics=("arbitrary",),
    ),
)


def kernel(inputs_, table, W, b):
    idx = inputs_.reshape(_NW, _CHUNKS_PER_W, _CHUNK).astype(jnp.int32)
    tail = _tail_pad(table)
    emb = _sc_gather()(idx, table, tail)
    x = _pool(emb.reshape(_B, _LCTX, _EP))
    return _proj(x, W, b.reshape(1, _V))
